# baseline (speedup 1.0000x reference)
import functools

import jax
import jax.numpy as jnp
from jax import lax
from jax.experimental import pallas as pl
from jax.experimental.pallas import tpu as pltpu
from jax.experimental.pallas import tpu_sc as plsc


ROWS = 512


def _add_kernel(x_ref, pos_ref, out_ref):
    out_ref[...] = x_ref[...] + pos_ref[...][:, None, :]


def _kernel_tc(x, pos_table):
    chunk, b, d = x.shape
    grid = (chunk // ROWS,)
    return pl.pallas_call(
        _add_kernel,
        grid=grid,
        in_specs=[
            pl.BlockSpec((ROWS, b, d), lambda i: (i, 0, 0)),
            pl.BlockSpec((ROWS, d), lambda i: (i, 0)),
        ],
        out_specs=pl.BlockSpec((ROWS, b, d), lambda i: (i, 0, 0)),
        out_shape=jax.ShapeDtypeStruct((chunk, b, d), x.dtype),
    )(x, pos_table[:chunk])


NW = 32
G = 4
NBUF = 6
LEAD = 3
LANES = 16
UNROLL = 4


def _kernel_sc(x, pos_table):
    chunk, b, d = x.shape
    per_w = chunk // NW
    steps = per_w // G

    mesh = plsc.VectorSubcoreMesh(core_axis_name="c", subcore_axis_name="s")

    scratch = (
        [pltpu.VMEM((G, d), jnp.float32) for _ in range(NBUF)]
        + [pltpu.VMEM((G, b, d), jnp.float32) for _ in range(NBUF)]
        + [pltpu.SemaphoreType.DMA for _ in range(2 * NBUF)]
    )

    @functools.partial(
        pl.kernel,
        mesh=mesh,
        out_type=jax.ShapeDtypeStruct((chunk, b, d), jnp.float32),
        scratch_types=scratch,
    )
    def k(x_hbm, pos_hbm, out_hbm, *bufs):
        pos_v = bufs[0:NBUF]
        x_v = bufs[NBUF:2 * NBUF]
        in_sem = bufs[2 * NBUF:3 * NBUF]
        out_sem = bufs[3 * NBUF:4 * NBUF]

        wid = lax.axis_index("s") * 2 + lax.axis_index("c")
        i_base = wid * per_w

        def issue_in(s, p):
            i0 = i_base + s * G
            pltpu.async_copy(pos_hbm.at[pl.ds(i0, G)], pos_v[p], in_sem[p])
            pltpu.async_copy(x_hbm.at[pl.ds(i0, G)], x_v[p], in_sem[p])

        def wait_in(p):
            pltpu.make_async_copy(pos_hbm.at[pl.ds(0, G)], pos_v[p],
                                  in_sem[p]).wait()
            pltpu.make_async_copy(x_hbm.at[pl.ds(0, G)], x_v[p],
                                  in_sem[p]).wait()

        def issue_out(s, p):
            i0 = i_base + s * G
            pltpu.async_copy(x_v[p], out_hbm.at[pl.ds(i0, G)], out_sem[p])

        def wait_out(p):
            pltpu.make_async_copy(x_v[p], out_hbm.at[pl.ds(0, G)],
                                  out_sem[p]).wait()

        def compute(p):
            pv_ref = pos_v[p]
            xv_ref = x_v[p]

            def body(t, c):
                g = t >> 4
                j4 = t & 15
                for u in range(UNROLL):
                    coff = (j4 * UNROLL + u) * LANES
                    pv = pv_ref[g, pl.ds(coff, LANES)]
                    for bb in range(b):
                        xv_ref[g, bb, pl.ds(coff, LANES)] = (
                            xv_ref[g, bb, pl.ds(coff, LANES)] + pv)
                return c

            lax.fori_loop(0, G * d // (LANES * UNROLL), body, 0)

        for s in range(LEAD):
            issue_in(s, s % NBUF)

        for s in range(LEAD):
            p = s % NBUF
            wait_in(p)
            compute(p)
            issue_out(s, p)
            r = (s + LEAD) % NBUF
            if s + LEAD - NBUF >= 0:
                wait_out(r)
            issue_in(s + LEAD, r)

        n_steady = ((steps - 2 * LEAD) // NBUF) * NBUF

        def steady(it, carry):
            for p0 in range(NBUF):
                s = LEAD + it * NBUF + p0
                p = (LEAD + p0) % NBUF
                wait_in(p)
                r = (p + LEAD) % NBUF
                wait_out(r)
                issue_in(s + LEAD, r)
                compute(p)
                issue_out(s, p)
            return carry

        lax.fori_loop(0, n_steady // NBUF, steady, 0)

        for s in range(LEAD + n_steady, steps):
            p = s % NBUF
            wait_in(p)
            compute(p)
            issue_out(s, p)
            if s + LEAD < steps:
                r = (p + LEAD) % NBUF
                if s + LEAD - NBUF >= 0:
                    wait_out(r)
                issue_in(s + LEAD, r)

        for p in range(NBUF):
            wait_out(p)

    return k(x, pos_table[:chunk])


def kernel(x, pos_table):
    return _kernel_sc(x, pos_table)


# speedup vs baseline: 1.0581x; 1.0581x over previous
import functools

import jax
import jax.numpy as jnp
from jax import lax
from jax.experimental import pallas as pl
from jax.experimental.pallas import tpu as pltpu
from jax.experimental.pallas import tpu_sc as plsc


ROWS = 512


def _add_kernel(x_ref, pos_ref, out_ref):
    out_ref[...] = x_ref[...] + pos_ref[...][:, None, :]


def _kernel_tc(x, pos_table):
    chunk, b, d = x.shape
    grid = (chunk // ROWS,)
    return pl.pallas_call(
        _add_kernel,
        grid=grid,
        in_specs=[
            pl.BlockSpec((ROWS, b, d), lambda i: (i, 0, 0)),
            pl.BlockSpec((ROWS, d), lambda i: (i, 0)),
        ],
        out_specs=pl.BlockSpec((ROWS, b, d), lambda i: (i, 0, 0)),
        out_shape=jax.ShapeDtypeStruct((chunk, b, d), x.dtype),
    )(x, pos_table[:chunk])


NW = 32
G = 4
NBUF = 6
LEAD = 3
LANES = 16
UNROLL = 4


def _kernel_sc(x, pos_table):
    chunk, b, d = x.shape
    per_w = chunk // NW
    steps = per_w // G

    mesh = plsc.VectorSubcoreMesh(core_axis_name="c", subcore_axis_name="s")

    scratch = (
        [pltpu.VMEM((G, d), jnp.float32) for _ in range(NBUF)]
        + [pltpu.VMEM((G, b, d), jnp.float32) for _ in range(NBUF)]
        + [pltpu.SemaphoreType.DMA for _ in range(2 * NBUF)]
    )

    @functools.partial(
        pl.kernel,
        mesh=mesh,
        out_type=jax.ShapeDtypeStruct((chunk, b, d), jnp.float32),
        scratch_types=scratch,
    )
    def k(x_hbm, pos_hbm, out_hbm, *bufs):
        pos_v = bufs[0:NBUF]
        x_v = bufs[NBUF:2 * NBUF]
        in_sem = bufs[2 * NBUF:3 * NBUF]
        out_sem = bufs[3 * NBUF:4 * NBUF]

        wid = lax.axis_index("s") * 2 + lax.axis_index("c")
        i_base = wid * per_w

        def issue_in(s, p):
            i0 = i_base + s * G
            pltpu.async_copy(pos_hbm.at[pl.ds(i0, G)], pos_v[p], in_sem[p])
            pltpu.async_copy(x_hbm.at[pl.ds(i0, G)], x_v[p], in_sem[p])

        def wait_in(p):
            pltpu.make_async_copy(pos_hbm.at[pl.ds(0, G)], pos_v[p],
                                  in_sem[p]).wait()
            pltpu.make_async_copy(x_hbm.at[pl.ds(0, G)], x_v[p],
                                  in_sem[p]).wait()

        def issue_out(s, p):
            i0 = i_base + s * G
            pltpu.async_copy(x_v[p], out_hbm.at[pl.ds(i0, G)], out_sem[p])

        def wait_out(p):
            pltpu.make_async_copy(x_v[p], out_hbm.at[pl.ds(0, G)],
                                  out_sem[p]).wait()

        def compute(p):
            pv_ref = pos_v[p]
            xv_ref = x_v[p]

            def body(t, c):
                g = t >> 4
                j4 = t & 15
                for u in range(UNROLL):
                    coff = (j4 * UNROLL + u) * LANES
                    pv = pv_ref[g, pl.ds(coff, LANES)]
                    for bb in range(b):
                        plsc.addupdate(xv_ref.at[g, bb, pl.ds(coff, LANES)],
                                       pv)
                return c

            lax.fori_loop(0, G * d // (LANES * UNROLL), body, 0)

        for s in range(LEAD):
            issue_in(s, s % NBUF)

        for s in range(LEAD):
            p = s % NBUF
            wait_in(p)
            compute(p)
            issue_out(s, p)
            r = (s + LEAD) % NBUF
            if s + LEAD - NBUF >= 0:
                wait_out(r)
            issue_in(s + LEAD, r)

        n_steady = ((steps - 2 * LEAD) // NBUF) * NBUF

        def steady(it, carry):
            for p0 in range(NBUF):
                s = LEAD + it * NBUF + p0
                p = (LEAD + p0) % NBUF
                wait_in(p)
                compute(p)
                issue_out(s, p)
                r = (p + LEAD) % NBUF
                wait_out(r)
                issue_in(s + LEAD, r)
            return carry

        lax.fori_loop(0, n_steady // NBUF, steady, 0)

        for s in range(LEAD + n_steady, steps):
            p = s % NBUF
            wait_in(p)
            compute(p)
            issue_out(s, p)
            if s + LEAD < steps:
                r = (p + LEAD) % NBUF
                if s + LEAD - NBUF >= 0:
                    wait_out(r)
                issue_in(s + LEAD, r)

        for p in range(NBUF):
            wait_out(p)

    return k(x, pos_table[:chunk])


def kernel(x, pos_table):
    return _kernel_sc(x, pos_table)


# speedup vs baseline: 1.0597x; 1.0015x over previous
import functools

import jax
import jax.numpy as jnp
from jax import lax
from jax.experimental import pallas as pl
from jax.experimental.pallas import tpu as pltpu
from jax.experimental.pallas import tpu_sc as plsc


ROWS = 512


def _add_kernel(x_ref, pos_ref, out_ref):
    out_ref[...] = x_ref[...] + pos_ref[...][:, None, :]


def _kernel_tc(x, pos_table):
    chunk, b, d = x.shape
    grid = (chunk // ROWS,)
    return pl.pallas_call(
        _add_kernel,
        grid=grid,
        in_specs=[
            pl.BlockSpec((ROWS, b, d), lambda i: (i, 0, 0)),
            pl.BlockSpec((ROWS, d), lambda i: (i, 0)),
        ],
        out_specs=pl.BlockSpec((ROWS, b, d), lambda i: (i, 0, 0)),
        out_shape=jax.ShapeDtypeStruct((chunk, b, d), x.dtype),
    )(x, pos_table[:chunk])


NW = 32
G = 4
NBUF = 6
LEAD = 4
LANES = 16
UNROLL = 4


def _kernel_sc(x, pos_table):
    chunk, b, d = x.shape
    per_w = chunk // NW
    steps = per_w // G

    mesh = plsc.VectorSubcoreMesh(core_axis_name="c", subcore_axis_name="s")

    scratch = (
        [pltpu.VMEM((G, d), jnp.float32) for _ in range(NBUF)]
        + [pltpu.VMEM((G, b, d), jnp.float32) for _ in range(NBUF)]
        + [pltpu.SemaphoreType.DMA for _ in range(2 * NBUF)]
    )

    @functools.partial(
        pl.kernel,
        mesh=mesh,
        out_type=jax.ShapeDtypeStruct((chunk, b, d), jnp.float32),
        scratch_types=scratch,
    )
    def k(x_hbm, pos_hbm, out_hbm, *bufs):
        pos_v = bufs[0:NBUF]
        x_v = bufs[NBUF:2 * NBUF]
        in_sem = bufs[2 * NBUF:3 * NBUF]
        out_sem = bufs[3 * NBUF:4 * NBUF]

        wid = lax.axis_index("s") * 2 + lax.axis_index("c")
        i_base = wid * per_w

        def issue_in(s, p):
            i0 = i_base + s * G
            pltpu.async_copy(pos_hbm.at[pl.ds(i0, G)], pos_v[p], in_sem[p])
            pltpu.async_copy(x_hbm.at[pl.ds(i0, G)], x_v[p], in_sem[p])

        def wait_in(p):
            pltpu.make_async_copy(pos_hbm.at[pl.ds(0, G)], pos_v[p],
                                  in_sem[p]).wait()
            pltpu.make_async_copy(x_hbm.at[pl.ds(0, G)], x_v[p],
                                  in_sem[p]).wait()

        def issue_out(s, p):
            i0 = i_base + s * G
            pltpu.async_copy(x_v[p], out_hbm.at[pl.ds(i0, G)], out_sem[p])

        def wait_out(p):
            pltpu.make_async_copy(x_v[p], out_hbm.at[pl.ds(0, G)],
                                  out_sem[p]).wait()

        def compute(p):
            pv_ref = pos_v[p]
            xv_ref = x_v[p]

            def body(t, c):
                g = t >> 4
                j4 = t & 15
                for u in range(UNROLL):
                    coff = (j4 * UNROLL + u) * LANES
                    pv = pv_ref[g, pl.ds(coff, LANES)]
                    for bb in range(b):
                        plsc.addupdate(xv_ref.at[g, bb, pl.ds(coff, LANES)],
                                       pv)
                return c

            lax.fori_loop(0, G * d // (LANES * UNROLL), body, 0)

        for s in range(LEAD):
            issue_in(s, s % NBUF)

        for s in range(LEAD):
            p = s % NBUF
            wait_in(p)
            compute(p)
            issue_out(s, p)
            r = (s + LEAD) % NBUF
            if s + LEAD - NBUF >= 0:
                wait_out(r)
            issue_in(s + LEAD, r)

        n_steady = ((steps - 2 * LEAD) // NBUF) * NBUF

        def steady(it, carry):
            for p0 in range(NBUF):
                s = LEAD + it * NBUF + p0
                p = (LEAD + p0) % NBUF
                wait_in(p)
                compute(p)
                issue_out(s, p)
                r = (p + LEAD) % NBUF
                wait_out(r)
                issue_in(s + LEAD, r)
            return carry

        lax.fori_loop(0, n_steady // NBUF, steady, 0)

        for s in range(LEAD + n_steady, steps):
            p = s % NBUF
            wait_in(p)
            compute(p)
            issue_out(s, p)
            if s + LEAD < steps:
                r = (p + LEAD) % NBUF
                if s + LEAD - NBUF >= 0:
                    wait_out(r)
                issue_in(s + LEAD, r)

        for p in range(NBUF):
            wait_out(p)

    return k(x, pos_table[:chunk])


def kernel(x, pos_table):
    return _kernel_sc(x, pos_table)
